# baseline (speedup 1.0000x reference)
import functools
import math

import jax
import jax.numpy as jnp
from jax import lax
from jax.experimental import pallas as pl

_B = 16384
_STATUS = 128
_QL = 50
_EF = 16
_H = 4
_DH = 32
_GF = 1600
_HID = 64
_OUT = 64

_HI = lax.Precision.HIGHEST
_DP = lax.Precision.DEFAULT


def _gumbel_noise(bsz):
    u = jax.random.uniform(jax.random.key(42), (bsz, _QL), dtype=jnp.float32)
    return -jnp.log(-jnp.log(u + 1e-20) + 1e-20)


def _body(vs_ref, img_ref, g_ref, wq_ref, wkt_ref, w1_ref, b1_ref, w2_ref,
          b2_ref, out_ref):
    bt = vs_ref.shape[0]
    img = img_ref[...].reshape(bt, _GF)
    vs = vs_ref[...]
    status = vs[:, :_STATUS]
    em = vs[:, _STATUS:]

    qf = jnp.dot(status, wq_ref[...], precision=_DP)
    t = jnp.dot(qf, wkt_ref[...], precision=_DP) / jnp.float32(
        _H * math.sqrt(_DH))

    col_f = lax.broadcasted_iota(jnp.int32, (_EF, _QL * _EF), 1)
    row_f = lax.broadcasted_iota(jnp.int32, (_EF, _QL * _EF), 0)
    rep_t = (lax.rem(col_f, _EF) == row_f).astype(jnp.float32)

    col_q = lax.broadcasted_iota(jnp.int32, (_QL * _EF, _QL), 0)
    q_q = lax.broadcasted_iota(jnp.int32, (_QL * _EF, _QL), 1)
    seg = (col_q // _EF == q_q).astype(jnp.float32)

    t_rep = jnp.dot(t, rep_t, precision=_DP)
    wm = jnp.dot(em * t_rep, seg, precision=_DP)

    nz = (em != 0.0).astype(jnp.float32)
    cnt = jnp.dot(nz, seg, precision=_DP)
    wm = jnp.where(cnt == 0.0, jnp.float32(-1e8), wm)

    logits = (wm + jnp.float32(1e-8) + g_ref[...]) / jnp.float32(0.1)
    m = jnp.max(logits, axis=-1, keepdims=True)
    e = jnp.exp(logits - m)
    sel = e / jnp.sum(e, axis=-1, keepdims=True)

    sel_rep = jnp.dot(sel, seg.T, precision=_DP)
    selected = jnp.dot(em * sel_rep, rep_t.T, precision=_DP)

    w1 = w1_ref[...]
    h = (jnp.dot(status, w1[:_STATUS], precision=_DP)
         + jnp.dot(selected, w1[_STATUS:_STATUS + _EF], precision=_DP)
         + jnp.dot(img, w1[_STATUS + _EF:], precision=_DP)
         + b1_ref[...])
    h = jnp.maximum(h, 0.0)
    out = jnp.maximum(jnp.dot(h, w2_ref[...], precision=_DP) + b2_ref[...], 0.0)
    out_ref[...] = out


@jax.jit
def kernel(vector_state, image_state, Wq, Wk, Wv, W1, b1, W2, b2):
    del Wv
    bsz = vector_state.shape[0]
    g = _gumbel_noise(bsz)
    wkt = Wk.T
    b1r = b1.reshape(1, _HID)
    b2r = b2.reshape(1, _OUT)

    bt = 512
    grid = (bsz // bt,)
    row = lambda i: (i, 0)
    rep = lambda i: (0, 0)
    return pl.pallas_call(
        _body,
        grid=grid,
        in_specs=[
            pl.BlockSpec((bt, _STATUS + _QL * _EF), row),
            pl.BlockSpec((bt,) + image_state.shape[1:],
                         lambda i: (i, 0, 0, 0)),
            pl.BlockSpec((bt, _QL), row),
            pl.BlockSpec(Wq.shape, rep),
            pl.BlockSpec(wkt.shape, rep),
            pl.BlockSpec(W1.shape, rep),
            pl.BlockSpec(b1r.shape, rep),
            pl.BlockSpec(W2.shape, rep),
            pl.BlockSpec(b2r.shape, rep),
        ],
        out_specs=pl.BlockSpec((bt, _OUT), row),
        out_shape=jax.ShapeDtypeStruct((bsz, _OUT), jnp.float32),
    )(vector_state, image_state, g, Wq, wkt, W1, b1r, W2, b2r)


# speedup vs baseline: 1.2402x; 1.2402x over previous
import functools
import math

import jax
import jax.numpy as jnp
from jax import lax
from jax.experimental import pallas as pl

_B = 16384
_STATUS = 128
_QL = 50
_EF = 16
_H = 4
_DH = 32
_GF = 1600
_HID = 64
_OUT = 64

_HI = lax.Precision.HIGHEST
_DP = lax.Precision.DEFAULT


def _gumbel_noise(bsz):
    u = jax.random.uniform(jax.random.key(42), (bsz, _QL), dtype=jnp.float32)
    return -jnp.log(-jnp.log(u + 1e-20) + 1e-20)


def _body(vs_ref, img_ref, g_ref, wq_ref, wkt_ref, w1_ref, b1_ref, w2_ref,
          b2_ref, out_ref):
    bt = vs_ref.shape[0]
    img = img_ref[...].reshape(bt, _GF)
    vs = vs_ref[...]
    status = vs[:, :_STATUS]
    em = vs[:, _STATUS:]

    qf = jnp.dot(status, wq_ref[...], precision=_DP)
    t = jnp.dot(qf, wkt_ref[...], precision=_DP) / jnp.float32(
        _H * math.sqrt(_DH))

    col_f = lax.broadcasted_iota(jnp.int32, (_EF, _QL * _EF), 1)
    row_f = lax.broadcasted_iota(jnp.int32, (_EF, _QL * _EF), 0)
    rep_t = (lax.rem(col_f, _EF) == row_f).astype(jnp.float32)

    col_q = lax.broadcasted_iota(jnp.int32, (_QL * _EF, _QL), 0)
    q_q = lax.broadcasted_iota(jnp.int32, (_QL * _EF, _QL), 1)
    seg = (col_q // _EF == q_q).astype(jnp.float32)

    t_rep = jnp.dot(t, rep_t, precision=_DP)
    wm = jnp.dot(em * t_rep, seg, precision=_DP)

    nz = (em != 0.0).astype(jnp.float32)
    cnt = jnp.dot(nz, seg, precision=_DP)
    wm = jnp.where(cnt == 0.0, jnp.float32(-1e8), wm)

    logits = (wm + jnp.float32(1e-8) + g_ref[...]) / jnp.float32(0.1)
    m = jnp.max(logits, axis=-1, keepdims=True)
    e = jnp.exp(logits - m)
    sel = e / jnp.sum(e, axis=-1, keepdims=True)

    sel_rep = jnp.dot(sel, seg.T, precision=_DP)
    selected = jnp.dot(em * sel_rep, rep_t.T, precision=_DP)

    w1 = w1_ref[...]
    h = (jnp.dot(status, w1[:_STATUS], precision=_DP)
         + jnp.dot(selected, w1[_STATUS:_STATUS + _EF], precision=_DP)
         + jnp.dot(img, w1[_STATUS + _EF:], precision=_DP)
         + b1_ref[...])
    h = jnp.maximum(h, 0.0)
    out = jnp.maximum(jnp.dot(h, w2_ref[...], precision=_DP) + b2_ref[...], 0.0)
    out_ref[...] = out


@jax.jit
def kernel(vector_state, image_state, Wq, Wk, Wv, W1, b1, W2, b2):
    del Wv
    bsz = vector_state.shape[0]
    img2 = image_state.reshape(bsz, 40, 40)
    g = _gumbel_noise(bsz)
    wkt = Wk.T
    b1r = b1.reshape(1, _HID)
    b2r = b2.reshape(1, _OUT)

    bt = 512
    grid = (bsz // bt,)
    row = lambda i: (i, 0)
    rep = lambda i: (0, 0)
    return pl.pallas_call(
        _body,
        grid=grid,
        in_specs=[
            pl.BlockSpec((bt, _STATUS + _QL * _EF), row),
            pl.BlockSpec((bt, 40, 40), lambda i: (i, 0, 0)),
            pl.BlockSpec((bt, _QL), row),
            pl.BlockSpec(Wq.shape, rep),
            pl.BlockSpec(wkt.shape, rep),
            pl.BlockSpec(W1.shape, rep),
            pl.BlockSpec(b1r.shape, rep),
            pl.BlockSpec(W2.shape, rep),
            pl.BlockSpec(b2r.shape, rep),
        ],
        out_specs=pl.BlockSpec((bt, _OUT), row),
        out_shape=jax.ShapeDtypeStruct((bsz, _OUT), jnp.float32),
    )(vector_state, img2, g, Wq, wkt, W1, b1r, W2, b2r)


# speedup vs baseline: 4.6476x; 3.7474x over previous
import math

import jax
import jax.numpy as jnp
from jax import lax
from jax.experimental import pallas as pl

_STATUS = 128
_QL = 50
_EF = 16
_EMD = _QL * _EF
_H = 4
_DH = 32
_GF = 1600
_HID = 64
_OUT = 64

_DP = lax.Precision.DEFAULT


def _gumbel_noise(bsz):
    u = jax.random.uniform(jax.random.key(42), (bsz, _QL), dtype=jnp.float32)
    return -jnp.log(-jnp.log(u + 1e-20) + 1e-20)


def _body(vs_ref, img_ref, g_ref, wqt_ref, wk_ref, w1t_ref, b1_ref, w2t_ref,
          b2_ref, out_ref):
    vst = vs_ref[...]
    status_t = vst[:_STATUS]
    em_t = vst[_STATUS:]

    qf_t = jnp.dot(wqt_ref[...], status_t, precision=_DP)
    t_t = jnp.dot(wk_ref[...], qf_t, precision=_DP) / jnp.float32(
        _H * math.sqrt(_DH))

    col_f = lax.broadcasted_iota(jnp.int32, (_EMD, _EF), 0)
    row_f = lax.broadcasted_iota(jnp.int32, (_EMD, _EF), 1)
    rep = (lax.rem(col_f, _EF) == row_f).astype(jnp.float32)

    q_c = lax.broadcasted_iota(jnp.int32, (_QL, _EMD), 1)
    q_r = lax.broadcasted_iota(jnp.int32, (_QL, _EMD), 0)
    seg = (q_c // _EF == q_r).astype(jnp.float32)

    trep_t = jnp.dot(rep, t_t, precision=_DP)
    wm_t = jnp.dot(seg, em_t * trep_t, precision=_DP)

    nz = (em_t != 0.0).astype(jnp.float32)
    cnt = jnp.dot(seg, nz, precision=_DP)
    wm_t = jnp.where(cnt == 0.0, jnp.float32(-1e8), wm_t)

    logits = (wm_t + jnp.float32(1e-8) + g_ref[...]) / jnp.float32(0.1)
    m = jnp.max(logits, axis=0, keepdims=True)
    e = jnp.exp(logits - m)
    sel = e / jnp.sum(e, axis=0, keepdims=True)

    selrep_t = jnp.dot(seg.T, sel, precision=_DP)
    selected_t = jnp.dot(rep.T, em_t * selrep_t, precision=_DP)

    w1t = w1t_ref[...]
    h = (jnp.dot(w1t[:, :_STATUS], status_t, precision=_DP)
         + jnp.dot(w1t[:, _STATUS:_STATUS + _EF], selected_t, precision=_DP)
         + jnp.dot(w1t[:, _STATUS + _EF:], img_ref[...], precision=_DP)
         + b1_ref[...])
    h = jnp.maximum(h, 0.0)
    out = jnp.maximum(
        jnp.dot(w2t_ref[...], h, precision=_DP) + b2_ref[...], 0.0)
    out_ref[...] = out


@jax.jit
def kernel(vector_state, image_state, Wq, Wk, Wv, W1, b1, W2, b2):
    del Wv
    bsz = vector_state.shape[0]
    vst = vector_state.T
    imgt = image_state.transpose(1, 2, 3, 0).reshape(_GF, bsz)
    gt = _gumbel_noise(bsz).T
    wqt = Wq.T
    w1t = W1.T
    w2t = W2.T
    b1c = b1.reshape(_HID, 1)
    b2c = b2.reshape(_OUT, 1)

    btl = 512
    grid = (bsz // btl,)
    col = lambda i: (0, i)
    rep = lambda i: (0, 0)
    out_t = pl.pallas_call(
        _body,
        grid=grid,
        in_specs=[
            pl.BlockSpec((_STATUS + _EMD, btl), col),
            pl.BlockSpec((_GF, btl), col),
            pl.BlockSpec((_QL, btl), col),
            pl.BlockSpec(wqt.shape, rep),
            pl.BlockSpec(Wk.shape, rep),
            pl.BlockSpec(w1t.shape, rep),
            pl.BlockSpec(b1c.shape, rep),
            pl.BlockSpec(w2t.shape, rep),
            pl.BlockSpec(b2c.shape, rep),
        ],
        out_specs=pl.BlockSpec((_OUT, btl), col),
        out_shape=jax.ShapeDtypeStruct((_OUT, bsz), jnp.float32),
    )(vst, imgt, gt, wqt, Wk, w1t, b1c, w2t, b2c)
    return out_t.T
